# baseline (speedup 1.0000x reference)
import jax
import jax.numpy as jnp
from jax.experimental import pallas as pl


def _sqrt(x):
    return jnp.where(x == 0.0, 0.0, x * jax.lax.rsqrt(x))


def _dot_bf16(a, b):
    return jnp.dot(a.astype(jnp.bfloat16), b.astype(jnp.bfloat16),
                   preferred_element_type=jnp.float32)


def _body(z_ref, in_v_ref, in_g_ref, in_b_ref, out_v_ref, out_g_ref,
          out_b_ref, cb_ref, zq_out_ref, idx_ref, ze_ref):
    f32 = jnp.float32
    zt = z_ref[0]
    in_v = in_v_ref[...]
    w_in = in_g_ref[...] * in_v / _sqrt(
        jnp.sum(in_v * in_v, axis=1, keepdims=True))
    z_e = _dot_bf16(w_in, zt) + in_b_ref[...]
    ze_ref[0] = z_e

    enc = z_e.T
    enc_n = enc / jnp.maximum(
        _sqrt(jnp.sum(enc * enc, axis=1, keepdims=True)), 1e-12)
    cb = cb_ref[...]
    cb_n = cb / jnp.maximum(
        _sqrt(jnp.sum(cb * cb, axis=1, keepdims=True)), 1e-12)
    csq = jnp.sum(cb_n * cb_n, axis=1)[None, :]
    a = jnp.sum(enc_n * enc_n, axis=1, keepdims=True)
    s = _dot_bf16(enc_n, cb_n.T)
    nd = -((a - 2.0 * s) + csq)
    cb_size = nd.shape[1]
    iota = jax.lax.broadcasted_iota(jnp.int32, nd.shape, 1)
    W = 4096
    st = None
    for w in range(0, cb_size, W):
        ndw = nd[:, w:w + W]
        vw = jnp.max(ndw, axis=1, keepdims=True)
        iw = jnp.min(jnp.where(ndw == vw, iota[:, w:w + W], cb_size),
                     axis=1, keepdims=True)
        vw_r = vw.astype(jnp.bfloat16).astype(f32)
        if st is None:
            st, si = vw_r, iw
        else:
            take = vw > st
            st = jnp.where(take, vw_r, st)
            si = jnp.where(take, iw, si)
    idx = si[:, 0]
    idx_ref[0, 0] = idx

    onehot = (iota == idx[:, None]).astype(f32)
    z_q = _dot_bf16(onehot, cb)
    out_v = out_v_ref[...]
    w_out = out_g_ref[...] * out_v / _sqrt(
        jnp.sum(out_v * out_v, axis=1, keepdims=True))
    zq_out_ref[0] = _dot_bf16(w_out, z_q.T) + out_b_ref[...]


def kernel(z, in_v, in_g, in_b, out_v, out_g, out_b, codebook):
    B, D, T = z.shape
    CB, CD = codebook.shape
    TT = 256
    grid = (B, T // TT)

    full = lambda shape: pl.BlockSpec(shape, lambda b, t: (0,) * len(shape))
    out_shapes = (
        jax.ShapeDtypeStruct((B, D, T), jnp.float32),
        jax.ShapeDtypeStruct((B, 1, T), jnp.int32),
        jax.ShapeDtypeStruct((B, CD, T), jnp.float32),
    )
    zq_out, idx3, z_e = pl.pallas_call(
        _body,
        grid=grid,
        in_specs=[
            pl.BlockSpec((1, D, TT), lambda b, t: (b, 0, t)),
            full((CD, D)),
            full((CD, 1)),
            full((CD, 1)),
            full((D, CD)),
            full((D, 1)),
            full((D, 1)),
            full((CB, CD)),
        ],
        out_specs=[
            pl.BlockSpec((1, D, TT), lambda b, t: (b, 0, t)),
            pl.BlockSpec((1, 1, TT), lambda b, t: (b, 0, t)),
            pl.BlockSpec((1, CD, TT), lambda b, t: (b, 0, t)),
        ],
        out_shape=out_shapes,
    )(z, in_v, in_g[:, None], in_b[:, None], out_v, out_g[:, None],
      out_b[:, None], codebook)

    indices = idx3.reshape(B, T)
    zeros = jnp.zeros((B,), dtype=jnp.float32)
    return (zq_out, zeros, zeros, indices, z_e)


# speedup vs baseline: 1.2520x; 1.2520x over previous
import jax
import jax.numpy as jnp
from jax.experimental import pallas as pl

F32 = jnp.float32
BF16 = jnp.bfloat16


def _sqrt(x):
    return jnp.where(x == 0.0, 0.0, x * jax.lax.rsqrt(x))


def _prep_body(cbT_ref, cb_ref, in_v_ref, in_g_ref, out_v_ref, out_g_ref,
               cb_nT_ref, csq_ref, cb_bf_ref, w_in_ref, w_out_ref):
    cbT = cbT_ref[...]
    norm = jnp.maximum(_sqrt(jnp.sum(cbT * cbT, axis=0, keepdims=True)), 1e-12)
    cb_nT = cbT / norm
    cb_nT_ref[...] = cb_nT.astype(BF16)
    csq_ref[...] = jnp.sum(cb_nT * cb_nT, axis=0, keepdims=True)
    cb_bf_ref[...] = cb_ref[...].astype(BF16)
    in_v = in_v_ref[...]
    w_in = in_g_ref[...] * in_v / _sqrt(
        jnp.sum(in_v * in_v, axis=1, keepdims=True))
    w_in_ref[...] = w_in.astype(BF16)
    out_v = out_v_ref[...]
    w_out = out_g_ref[...] * out_v / _sqrt(
        jnp.sum(out_v * out_v, axis=1, keepdims=True))
    w_out_ref[...] = w_out.astype(BF16)


def _dg(a, b, dims):
    return jax.lax.dot_general(a, b, (dims, ((), ())),
                               preferred_element_type=F32)


def _main_body(z_ref, in_b_ref, out_b_ref, w_in_ref, w_out_ref, cb_nT_ref,
               csq_ref, cb_bf_ref, zq_out_ref, idx_ref, ze_ref):
    zt = z_ref[0]
    z_e = _dg(w_in_ref[...], zt.astype(BF16), (((1,), (0,)))) + in_b_ref[...]
    ze_ref[0] = z_e

    ssq = jnp.sum(z_e * z_e, axis=0, keepdims=True)
    r = jnp.maximum(_sqrt(ssq), 1e-12)
    enc2T = ((z_e / r) * 2.0).astype(BF16)
    s2 = _dg(enc2T, cb_nT_ref[...], (((0,), (0,))))
    a = jnp.sum((z_e / r) * (z_e / r), axis=0, keepdims=True).T
    nd = (s2 - a) - csq_ref[...]

    cb_size = nd.shape[1]
    iota = jax.lax.broadcasted_iota(jnp.int32, nd.shape, 1)
    W = 4096
    st = None
    for w in range(0, cb_size, W):
        ndw = nd[:, w:w + W]
        vw = jnp.max(ndw, axis=1, keepdims=True)
        iw = jnp.min(jnp.where(ndw == vw, iota[:, w:w + W], cb_size),
                     axis=1, keepdims=True)
        vw_r = vw.astype(BF16).astype(F32)
        if st is None:
            st, si = vw_r, iw
        else:
            take = vw > st
            st = jnp.where(take, vw_r, st)
            si = jnp.where(take, iw, si)
    idx = si[:, 0]
    idx_ref[0, 0] = idx

    onehot = (iota == si).astype(BF16)
    z_q = _dg(onehot, cb_bf_ref[...], (((1,), (0,))))
    zq_out_ref[0] = (_dg(w_out_ref[...], z_q.astype(BF16), (((1,), (1,))))
                     + out_b_ref[...])


def kernel(z, in_v, in_g, in_b, out_v, out_g, out_b, codebook):
    B, D, T = z.shape
    CB, CD = codebook.shape
    TT = 512
    grid = (B, T // TT)

    cbT = codebook.T

    prep_out = pl.pallas_call(
        _prep_body,
        out_shape=(
            jax.ShapeDtypeStruct((CD, CB), BF16),
            jax.ShapeDtypeStruct((1, CB), F32),
            jax.ShapeDtypeStruct((CB, CD), BF16),
            jax.ShapeDtypeStruct((CD, D), BF16),
            jax.ShapeDtypeStruct((D, CD), BF16),
        ),
    )(cbT, codebook, in_v, in_g[:, None], out_v, out_g[:, None])
    cb_nT, csq, cb_bf, w_in_bf, w_out_bf = prep_out

    full = lambda shape: pl.BlockSpec(shape, lambda b, t: (0,) * len(shape))
    out_shapes = (
        jax.ShapeDtypeStruct((B, D, T), F32),
        jax.ShapeDtypeStruct((B, 1, T), jnp.int32),
        jax.ShapeDtypeStruct((B, CD, T), F32),
    )
    zq_out, idx3, z_e = pl.pallas_call(
        _main_body,
        grid=grid,
        in_specs=[
            pl.BlockSpec((1, D, TT), lambda b, t: (b, 0, t)),
            full((CD, 1)),
            full((D, 1)),
            full((CD, D)),
            full((D, CD)),
            full((CD, CB)),
            full((1, CB)),
            full((CB, CD)),
        ],
        out_specs=[
            pl.BlockSpec((1, D, TT), lambda b, t: (b, 0, t)),
            pl.BlockSpec((1, 1, TT), lambda b, t: (b, 0, t)),
            pl.BlockSpec((1, CD, TT), lambda b, t: (b, 0, t)),
        ],
        out_shape=out_shapes,
    )(z, in_b[:, None], out_b[:, None], w_in_bf, w_out_bf, cb_nT, csq, cb_bf)

    indices = idx3.reshape(B, T)
    zeros = jnp.zeros((B,), dtype=F32)
    return (zq_out, zeros, zeros, indices, z_e)
